# baseline (speedup 1.0000x reference)
import functools

import jax
import jax.numpy as jnp
from jax import lax
from jax.experimental import pallas as pl
from jax.experimental.pallas import tpu as pltpu
from jax.experimental.pallas import tpu_sc as plsc

_N = 16384

_mesh = plsc.ScalarSubcoreMesh(axis_name="c", num_cores=1)


@functools.partial(
    pl.kernel,
    mesh=_mesh,
    out_type=jax.ShapeDtypeStruct((_N,), jnp.float32),
)
def _copy_kernel(a_hbm, out_hbm):
    pltpu.sync_copy(a_hbm, out_hbm)


def kernel(a):
    return _copy_kernel(a)


# speedup vs baseline: 12.7078x; 12.7078x over previous
import jax
import jax.numpy as jnp
from jax.experimental import pallas as pl


def _copy_body(a_ref, o_ref):
    o_ref[...] = a_ref[...]


def kernel(a):
    a2 = a.reshape(128, 128)
    out = pl.pallas_call(
        _copy_body,
        out_shape=jax.ShapeDtypeStruct((128, 128), jnp.float32),
    )(a2)
    return out.reshape(-1)
